# Initial kernel scaffold; baseline (speedup 1.0000x reference)
import jax
import jax.numpy as jnp
from jax.experimental import pallas as pl


def kernel(a, b, table):
    raise NotImplementedError("write your pallas kernel here")


# speedup vs baseline: 202.5197x; 202.5197x over previous
import functools

import jax
import jax.numpy as jnp
from jax import lax
from jax.experimental import pallas as pl
from jax.experimental.pallas import tpu as pltpu
from jax.experimental.pallas import tpu_sc as plsc

_NC = 2
_NS = 16
_NW = _NC * _NS
_NBUF = 2
_NSTEPS = 8


def _make_sc_lut(n):
    per_w = n // _NW
    chunk = per_w // _NSTEPS
    ochunk = chunk // 2
    nblk = chunk // 32
    assert per_w * _NW == n and chunk * _NSTEPS == per_w and nblk * 32 == chunk

    mesh = plsc.VectorSubcoreMesh(
        core_axis_name="c", subcore_axis_name="s",
        num_cores=_NC, num_subcores=_NS)

    @functools.partial(
        pl.kernel,
        out_type=jax.ShapeDtypeStruct((n // 2,), jnp.int32),
        mesh=mesh,
        compiler_params=pltpu.CompilerParams(needs_layout_passes=False),
        scratch_types=[
            pltpu.VMEM((256,), jnp.int32),
            [pltpu.VMEM((chunk,), jnp.int32) for _ in range(_NBUF)],
            [pltpu.VMEM((ochunk,), jnp.int32) for _ in range(_NBUF)],
            [pltpu.SemaphoreType.DMA for _ in range(_NBUF)],
            [pltpu.SemaphoreType.DMA for _ in range(_NBUF)],
        ],
    )
    def sc_lut(col_hbm, a_hbm, out_hbm, colv, abufs, obufs, isems, osems):
        wid = lax.axis_index("s") * _NC + lax.axis_index("c")
        base = wid * per_w
        obase = wid * (per_w // 2)

        pltpu.sync_copy(col_hbm, colv)

        def in_copy(g):
            k = g % _NBUF
            return pltpu.make_async_copy(
                a_hbm.at[pl.ds(base + g * chunk, chunk)], abufs[k], isems[k])

        def out_copy(g):
            k = g % _NBUF
            return pltpu.make_async_copy(
                obufs[k], out_hbm.at[pl.ds(obase + g * ochunk, ochunk)],
                osems[k])

        for g in range(_NBUF):
            in_copy(g).start()

        lane2 = lax.iota(jnp.int32, 16) * 2

        for g in range(_NSTEPS):
            k = g % _NBUF
            in_copy(g).wait()
            if g >= _NBUF:
                out_copy(g - _NBUF).wait()

            def blk(i, carry, _abuf=abufs[k], _obuf=obufs[k]):
                idx_e = lane2 + i * 32
                a_e = plsc.load_gather(_abuf, [idx_e])
                a_o = plsc.load_gather(_abuf, [idx_e + 1])
                g_e = plsc.load_gather(colv, [a_e + 128])
                g_o = plsc.load_gather(colv, [a_o + 128])
                _obuf[pl.ds(i * 16, 16)] = (g_e & 0xFFFF) | (g_o << 16)
                return carry

            lax.fori_loop(0, nblk, blk, 0, unroll=8)

            out_copy(g).start()
            if g + _NBUF < _NSTEPS:
                in_copy(g + _NBUF).start()

        for g in range(_NSTEPS - _NBUF, _NSTEPS):
            out_copy(g).wait()

    return sc_lut


def kernel(a, b, table):
    n = a.size
    column = jnp.take(table, b + 128, axis=1).astype(jnp.int32)
    a_flat = a.reshape(n)
    packed = _make_sc_lut(n)(column, a_flat)
    return lax.bitcast_convert_type(packed, jnp.int16).reshape(a.shape)


# speedup vs baseline: 245.8616x; 1.2140x over previous
import functools

import jax
import jax.numpy as jnp
from jax import lax
from jax.experimental import pallas as pl
from jax.experimental.pallas import tpu as pltpu
from jax.experimental.pallas import tpu_sc as plsc

_NC = 2
_NS = 16
_NW = _NC * _NS
_NBUF = 2
_NSTEPS = 8


def _make_sc_lut(n):
    per_w = n // _NW
    chunk = per_w // _NSTEPS
    ochunk = chunk // 2
    nblk = chunk // 32
    assert per_w * _NW == n and chunk * _NSTEPS == per_w and nblk * 32 == chunk

    mesh = plsc.VectorSubcoreMesh(
        core_axis_name="c", subcore_axis_name="s",
        num_cores=_NC, num_subcores=_NS)

    @functools.partial(
        pl.kernel,
        out_type=jax.ShapeDtypeStruct((n // 2,), jnp.int32),
        mesh=mesh,
        compiler_params=pltpu.CompilerParams(needs_layout_passes=False),
        scratch_types=[
            pltpu.VMEM((256,), jnp.int32),
            [pltpu.VMEM((chunk,), jnp.int32) for _ in range(_NBUF)],
            [pltpu.VMEM((ochunk,), jnp.int32) for _ in range(_NBUF)],
            [pltpu.SemaphoreType.DMA for _ in range(_NBUF)],
            [pltpu.SemaphoreType.DMA for _ in range(_NBUF)],
        ],
    )
    def sc_lut(col_hbm, a_hbm, out_hbm, colv, abufs, obufs, isems, osems):
        wid = lax.axis_index("s") * _NC + lax.axis_index("c")
        base = wid * per_w
        obase = wid * (per_w // 2)

        pltpu.sync_copy(col_hbm, colv)

        def in_copy(g):
            k = g % _NBUF
            return pltpu.make_async_copy(
                a_hbm.at[pl.ds(base + g * chunk, chunk)], abufs[k], isems[k])

        def out_copy(g):
            k = g % _NBUF
            return pltpu.make_async_copy(
                obufs[k], out_hbm.at[pl.ds(obase + g * ochunk, ochunk)],
                osems[k])

        for g in range(_NBUF):
            in_copy(g).start()

        lane2 = lax.iota(jnp.int32, 16) * 2

        for g in range(_NSTEPS):
            k = g % _NBUF
            in_copy(g).wait()
            if g >= _NBUF:
                out_copy(g - _NBUF).wait()

            abuf, obuf = abufs[k], obufs[k]

            @plsc.parallel_loop(0, nblk, step=1, unroll=8)
            def _blk(i):
                idx_e = lane2 + i * 32
                a_e = plsc.load_gather(abuf, [idx_e])
                a_o = plsc.load_gather(abuf, [idx_e + 1])
                g_e = plsc.load_gather(colv, [a_e + 128])
                g_o = plsc.load_gather(colv, [a_o + 128])
                obuf[pl.ds(i * 16, 16)] = (g_e & 0xFFFF) | (g_o << 16)

            out_copy(g).start()
            if g + _NBUF < _NSTEPS:
                in_copy(g + _NBUF).start()

        for g in range(_NSTEPS - _NBUF, _NSTEPS):
            out_copy(g).wait()

    return sc_lut


def kernel(a, b, table):
    n = a.size
    column = jnp.take(table, b + 128, axis=1).astype(jnp.int32)
    a_flat = a.reshape(n)
    packed = _make_sc_lut(n)(column, a_flat)
    return lax.bitcast_convert_type(packed, jnp.int16).reshape(a.shape)
